# baseline (speedup 1.0000x reference)
import functools

import jax
import jax.numpy as jnp
from jax.experimental import pallas as pl
from jax.experimental.pallas import tpu as pltpu

T, D, H, DH, F, E, K = 2048, 1024, 16, 64, 4096, 8, 2
BLK = 128
NB = (T * K) // BLK + E
P = NB * BLK
EPAD = 128
FT = 1024
NF = F // FT
GR = 16
GC = 8


def _recip(x):
    r = 1.0 / x
    return r * (2.0 - x * r)


def _rsqrt(x):
    r = jax.lax.rsqrt(x)
    return r * (1.5 - 0.5 * x * r * r)


def _qkv_body(x_ref, w_ref, b_ref, o_ref):
    o_ref[...] = jax.lax.dot_general(
        x_ref[...], w_ref[...], (((1,), (1,)), ((), ())),
        preferred_element_type=jnp.float32) + b_ref[0, :][None, :]


def _attn_body(q_ref, k_ref, v_ref, o_ref):
    scale = DH ** -0.5
    for half in range(2):
        sl = slice(half * DH, (half + 1) * DH)
        q = q_ref[:, sl]
        k = k_ref[:, sl]
        v = v_ref[:, sl]
        s = jax.lax.dot_general(q, k, (((1,), (1,)), ((), ())),
                                preferred_element_type=jnp.float32) * scale
        m = jnp.max(s, axis=1, keepdims=True)
        e = jnp.exp(s - m)
        p = e * _recip(jnp.sum(e, axis=1, keepdims=True))
        o_ref[:, sl] = jax.lax.dot_general(
            p, v, (((1,), (0,)), ((), ())),
            preferred_element_type=jnp.float32)


def _ln1_router_body(a_ref, wo_ref, bo_ref, x_ref, g_ref, b_ref, wg_ref,
                     x1_ref, ti_ref, gt_ref, st_ref):
    y = jax.lax.dot_general(a_ref[...], wo_ref[...], (((1,), (1,)), ((), ())),
                            preferred_element_type=jnp.float32)
    y = y + bo_ref[0, :][None, :] + x_ref[...]
    mu = jnp.mean(y, axis=1, keepdims=True)
    yc = y - mu
    var = jnp.mean(yc * yc, axis=1, keepdims=True)
    x1 = yc * _rsqrt(var + 1e-5) * g_ref[0, :][None, :] + b_ref[0, :][None, :]
    x1_ref[...] = x1

    logits = jax.lax.dot_general(x1, wg_ref[...], (((1,), (1,)), ((), ())),
                                 preferred_element_type=jnp.float32)
    col = jax.lax.broadcasted_iota(jnp.int32, logits.shape, 1)
    valid = col < E
    logits = jnp.where(valid, logits, jnp.float32(-1e30))
    l1 = jnp.max(logits, axis=1, keepdims=True)
    i1 = jnp.min(jnp.where(logits == l1, col, EPAD), axis=1, keepdims=True)
    lm = jnp.where(col == i1, jnp.float32(-1e30), logits)
    l2 = jnp.max(lm, axis=1, keepdims=True)
    i2 = jnp.min(jnp.where(lm == l2, col, EPAD), axis=1, keepdims=True)
    t = jnp.exp(l2 - l1)
    r = _recip(1.0 + t)
    ex = jnp.where(valid, jnp.exp(logits - l1), 0.0)
    probs = ex * _recip(jnp.sum(ex, axis=1, keepdims=True))
    ti_ref[...] = jnp.where(col == 0, i1, jnp.where(col == 1, i2, 0))
    gt_ref[...] = jnp.where(col == 0, r, jnp.where(col == 1, t * r, 0.0))

    mask = ((col == i1) | (col == i2)) & valid
    psum = jnp.sum(probs, axis=0, keepdims=True)
    fsum = jnp.sum(mask.astype(jnp.float32), axis=0, keepdims=True)

    @pl.when(pl.program_id(0) == 0)
    def _():
        st_ref[...] = jnp.zeros_like(st_ref)

    st_ref[0:1, :] += psum
    st_ref[1:2, :] += fsum


def _gather_body(tok_ref, *refs):
    o_ref = refs[-1]
    for i in range(GR):
        o_ref[i:i + 1, :] = refs[i][0]


def _ffn_body(be_ref, xg_ref, w1_ref, b1_ref, w2_ref, b2_ref, g_ref, o_ref):
    f = pl.program_id(1)
    h = jax.lax.dot_general(xg_ref[...], w1_ref[0], (((1,), (1,)), ((), ())),
                            preferred_element_type=jnp.float32)
    h = jnp.maximum(h + b1_ref[0, 0, :][None, :], 0.0)
    c = jax.lax.dot_general(h, w2_ref[0], (((1,), (1,)), ((), ())),
                            preferred_element_type=jnp.float32)

    @pl.when(f == 0)
    def _():
        o_ref[...] = jnp.zeros_like(o_ref)

    o_ref[...] += c

    @pl.when(f == NF - 1)
    def _():
        o_ref[...] = (o_ref[...] + b2_ref[0, 0, :][None, :]) * g_ref[0, 0, :][:, None]


def _combine_body(pos_ref, x1_ref, *refs):
    moe_refs = refs[:2 * GC]
    g_ref, b_ref, o_ref = refs[2 * GC], refs[2 * GC + 1], refs[2 * GC + 2]
    rows = [moe_refs[2 * r][0] + moe_refs[2 * r + 1][0] for r in range(GC)]
    y = x1_ref[...] + jnp.concatenate(rows, axis=0)
    mu = jnp.mean(y, axis=1, keepdims=True)
    yc = y - mu
    var = jnp.mean(yc * yc, axis=1, keepdims=True)
    o_ref[...] = yc * _rsqrt(var + 1e-5) * g_ref[0, :][None, :] + b_ref[0, :][None, :]


def kernel(x, Wqkv, bqkv, Wo, bo, ln1_g, ln1_b, ln2_g, ln2_b, wg, We1, be1, We2, be2):
    f32 = jnp.float32
    xf = x.reshape(T, D)

    qkv = pl.pallas_call(
        _qkv_body,
        grid=(6, 8),
        in_specs=[
            pl.BlockSpec((T // 8, D), lambda n, m: (m, 0)),
            pl.BlockSpec((3 * D // 6, D), lambda n, m: (n, 0)),
            pl.BlockSpec((1, 3 * D // 6), lambda n, m: (0, n)),
        ],
        out_specs=pl.BlockSpec((T // 8, 3 * D // 6), lambda n, m: (m, n)),
        out_shape=jax.ShapeDtypeStruct((T, 3 * D), f32),
    )(xf, Wqkv, bqkv.reshape(1, 3 * D))

    attn = pl.pallas_call(
        _attn_body,
        grid=(H // 2, 4),
        in_specs=[
            pl.BlockSpec((T // 4, 2 * DH), lambda h, m: (m, h)),
            pl.BlockSpec((T, 2 * DH), lambda h, m: (0, H // 2 + h)),
            pl.BlockSpec((T, 2 * DH), lambda h, m: (0, H + h)),
        ],
        out_specs=pl.BlockSpec((T // 4, 2 * DH), lambda h, m: (m, h)),
        out_shape=jax.ShapeDtypeStruct((T, D), f32),
    )(qkv, qkv, qkv)

    wg_pad = jnp.zeros((EPAD, D), f32).at[:E].set(wg)
    x1, ti, gt, st = pl.pallas_call(
        _ln1_router_body,
        grid=(8,),
        in_specs=[
            pl.BlockSpec((T // 8, D), lambda m: (m, 0)),
            pl.BlockSpec((D, D), lambda m: (0, 0)),
            pl.BlockSpec((1, D), lambda m: (0, 0)),
            pl.BlockSpec((T // 8, D), lambda m: (m, 0)),
            pl.BlockSpec((1, D), lambda m: (0, 0)),
            pl.BlockSpec((1, D), lambda m: (0, 0)),
            pl.BlockSpec((EPAD, D), lambda m: (0, 0)),
        ],
        out_specs=[
            pl.BlockSpec((T // 8, D), lambda m: (m, 0)),
            pl.BlockSpec((T // 8, EPAD), lambda m: (m, 0)),
            pl.BlockSpec((T // 8, EPAD), lambda m: (m, 0)),
            pl.BlockSpec((8, EPAD), lambda m: (0, 0)),
        ],
        out_shape=[
            jax.ShapeDtypeStruct((T, D), f32),
            jax.ShapeDtypeStruct((T, EPAD), jnp.int32),
            jax.ShapeDtypeStruct((T, EPAD), f32),
            jax.ShapeDtypeStruct((8, EPAD), f32),
        ],
    )(attn, Wo, bo.reshape(1, D), xf, ln1_g.reshape(1, D), ln1_b.reshape(1, D),
      wg_pad)

    topi = ti[:, :K]
    gates = gt[:, :K]
    aux = f32(E) * jnp.sum((st[1, :E] / T) * (st[0, :E] / T))

    e_flat = topi.reshape(-1)
    onehot = (e_flat[:, None] == jnp.arange(E, dtype=jnp.int32)[None, :]).astype(jnp.int32)
    rank = jnp.sum((jnp.cumsum(onehot, axis=0) - onehot) * onehot, axis=1)
    counts = jnp.sum(onehot, axis=0)
    nblk = (counts + BLK - 1) // BLK
    bstart = jnp.concatenate([jnp.zeros(1, jnp.int32), jnp.cumsum(nblk)])[:E]
    pos = (bstart * BLK)[e_flat] + rank
    tok_of_pair = jnp.arange(T * K, dtype=jnp.int32) // K
    row_token = jnp.zeros(P, jnp.int32).at[pos].set(tok_of_pair)
    row_gate = jnp.zeros(P, f32).at[pos].set(gates.reshape(-1))
    block_expert = (jnp.sum(jnp.arange(NB, dtype=jnp.int32)[:, None] >= bstart[None, :],
                            axis=1) - 1).astype(jnp.int32)

    def _gspec(i):
        return pl.BlockSpec((1, 1, D), lambda j, tok: (tok[j * GR + i], 0, 0))

    x1r = x1.reshape(T, 1, D)
    xg = pl.pallas_call(
        _gather_body,
        grid_spec=pltpu.PrefetchScalarGridSpec(
            num_scalar_prefetch=1,
            grid=(P // GR,),
            in_specs=[_gspec(i) for i in range(GR)],
            out_specs=pl.BlockSpec((GR, D), lambda j, tok: (j, 0)),
        ),
        out_shape=jax.ShapeDtypeStruct((P, D), f32),
    )(row_token, *([x1r] * GR))

    moe = pl.pallas_call(
        _ffn_body,
        grid_spec=pltpu.PrefetchScalarGridSpec(
            num_scalar_prefetch=1,
            grid=(NB, NF),
            in_specs=[
                pl.BlockSpec((BLK, D), lambda b, f, be: (b, 0)),
                pl.BlockSpec((1, FT, D), lambda b, f, be: (be[b], f, 0)),
                pl.BlockSpec((1, 1, FT), lambda b, f, be: (be[b], 0, f)),
                pl.BlockSpec((1, D, FT), lambda b, f, be: (be[b], 0, f)),
                pl.BlockSpec((1, 1, D), lambda b, f, be: (be[b], 0, 0)),
                pl.BlockSpec((1, 1, BLK), lambda b, f, be: (b, 0, 0)),
            ],
            out_specs=pl.BlockSpec((BLK, D), lambda b, f, be: (b, 0)),
        ),
        out_shape=jax.ShapeDtypeStruct((P, D), f32),
    )(block_expert, xg, We1, be1.reshape(E, 1, F), We2, be2.reshape(E, 1, D),
      row_gate.reshape(NB, 1, BLK))

    def _cspec(i):
        return pl.BlockSpec((1, 1, D), lambda j, ps: (ps[j * 2 * GC + i], 0, 0))

    out = pl.pallas_call(
        _combine_body,
        grid_spec=pltpu.PrefetchScalarGridSpec(
            num_scalar_prefetch=1,
            grid=(T // GC,),
            in_specs=[pl.BlockSpec((GC, D), lambda j, ps: (j, 0))]
            + [_cspec(i) for i in range(2 * GC)]
            + [pl.BlockSpec((1, D), lambda j, ps: (0, 0)),
               pl.BlockSpec((1, D), lambda j, ps: (0, 0))],
            out_specs=pl.BlockSpec((GC, D), lambda j, ps: (j, 0)),
        ),
        out_shape=jax.ShapeDtypeStruct((T, D), f32),
    )(pos.astype(jnp.int32), x1, *([moe.reshape(P, 1, D)] * (2 * GC)),
      ln2_g.reshape(1, D), ln2_b.reshape(1, D))

    return (out.reshape(1, T, D), aux)


# speedup vs baseline: 1.5103x; 1.5103x over previous
import functools

import jax
import jax.numpy as jnp
from jax import lax
from jax.experimental import pallas as pl
from jax.experimental.pallas import tpu as pltpu, tpu_sc as plsc

T, D, H, DH, F, E, K = 2048, 1024, 16, 64, 4096, 8, 2
BLK = 256
NB = (T * K) // BLK + E
P = NB * BLK
EPAD = 128
FT = 1024
NF = F // FT
GC = 256
NWORK = 32


def _sc_row_gather(B, nchunks):
    b_per_w = B // NWORK
    cb = b_per_w // nchunks
    mesh = plsc.VectorSubcoreMesh(core_axis_name="c", subcore_axis_name="s",
                                  num_cores=2, num_subcores=16)

    @functools.partial(
        pl.kernel, mesh=mesh,
        out_type=jax.ShapeDtypeStruct((B, D), jnp.float32),
        scratch_types=[
            pltpu.VMEM((cb,), jnp.int32),
            pltpu.VMEM((cb, D), jnp.float32),
            pltpu.SemaphoreType.DMA,
        ],
    )
    def k(table_hbm, idx_hbm, out_hbm, idx_v, rows_v, sem):
        wid = lax.axis_index("s") * 2 + lax.axis_index("c")
        for chunk in range(nchunks):
            base = wid * b_per_w + chunk * cb
            pltpu.sync_copy(idx_hbm.at[pl.ds(base, cb)], idx_v)
            pltpu.async_copy(table_hbm.at[idx_v], rows_v, sem).wait()
            pltpu.sync_copy(rows_v, out_hbm.at[pl.ds(base, cb)])

    return k


def _recip(x):
    r = 1.0 / x
    return r * (2.0 - x * r)


def _rsqrt(x):
    r = jax.lax.rsqrt(x)
    return r * (1.5 - 0.5 * x * r * r)


def _qkv_body(x_ref, w_ref, b_ref, o_ref):
    o_ref[...] = jax.lax.dot_general(
        x_ref[...], w_ref[...], (((1,), (1,)), ((), ())),
        preferred_element_type=jnp.float32) + b_ref[0, :][None, :]


def _attn_body(q_ref, k_ref, v_ref, o_ref):
    scale = DH ** -0.5
    for half in range(2):
        sl = slice(half * DH, (half + 1) * DH)
        q = q_ref[:, sl]
        k = k_ref[:, sl]
        v = v_ref[:, sl]
        s = jax.lax.dot_general(q, k, (((1,), (1,)), ((), ())),
                                preferred_element_type=jnp.float32) * scale
        m = jnp.max(s, axis=1, keepdims=True)
        e = jnp.exp(s - m)
        p = e * _recip(jnp.sum(e, axis=1, keepdims=True))
        o_ref[:, sl] = jax.lax.dot_general(
            p, v, (((1,), (0,)), ((), ())),
            preferred_element_type=jnp.float32)


def _ln1_router_body(a_ref, wo_ref, bo_ref, x_ref, g_ref, b_ref, wg_ref,
                     x1_ref, ti_ref, gt_ref, st_ref):
    y = jax.lax.dot_general(a_ref[...], wo_ref[...], (((1,), (1,)), ((), ())),
                            preferred_element_type=jnp.float32)
    y = y + bo_ref[0, :][None, :] + x_ref[...]
    mu = jnp.mean(y, axis=1, keepdims=True)
    yc = y - mu
    var = jnp.mean(yc * yc, axis=1, keepdims=True)
    x1 = yc * _rsqrt(var + 1e-5) * g_ref[0, :][None, :] + b_ref[0, :][None, :]
    x1_ref[...] = x1

    logits = jax.lax.dot_general(x1, wg_ref[...], (((1,), (1,)), ((), ())),
                                 preferred_element_type=jnp.float32)
    col = jax.lax.broadcasted_iota(jnp.int32, logits.shape, 1)
    valid = col < E
    logits = jnp.where(valid, logits, jnp.float32(-1e30))
    l1 = jnp.max(logits, axis=1, keepdims=True)
    i1 = jnp.min(jnp.where(logits == l1, col, EPAD), axis=1, keepdims=True)
    lm = jnp.where(col == i1, jnp.float32(-1e30), logits)
    l2 = jnp.max(lm, axis=1, keepdims=True)
    i2 = jnp.min(jnp.where(lm == l2, col, EPAD), axis=1, keepdims=True)
    t = jnp.exp(l2 - l1)
    r = _recip(1.0 + t)
    ex = jnp.where(valid, jnp.exp(logits - l1), 0.0)
    probs = ex * _recip(jnp.sum(ex, axis=1, keepdims=True))
    ti_ref[...] = jnp.where(col == 0, i1, jnp.where(col == 1, i2, 0))
    gt_ref[...] = jnp.where(col == 0, r, jnp.where(col == 1, t * r, 0.0))

    mask = ((col == i1) | (col == i2)) & valid
    psum = jnp.sum(probs, axis=0, keepdims=True)
    fsum = jnp.sum(mask.astype(jnp.float32), axis=0, keepdims=True)

    @pl.when(pl.program_id(0) == 0)
    def _():
        st_ref[...] = jnp.zeros_like(st_ref)

    st_ref[0:1, :] += psum
    st_ref[1:2, :] += fsum


def _ffn_body(be_ref, xg_ref, w1_ref, b1_ref, w2_ref, b2_ref, g_ref, o_ref):
    f = pl.program_id(1)
    h = jax.lax.dot_general(xg_ref[...], w1_ref[0], (((1,), (1,)), ((), ())),
                            preferred_element_type=jnp.float32)
    h = jnp.maximum(h + b1_ref[0, 0, :][None, :], 0.0)
    c = jax.lax.dot_general(h, w2_ref[0], (((1,), (1,)), ((), ())),
                            preferred_element_type=jnp.float32)

    @pl.when(f == 0)
    def _():
        o_ref[...] = jnp.zeros_like(o_ref)

    o_ref[...] += c

    @pl.when(f == NF - 1)
    def _():
        o_ref[...] = (o_ref[...] + b2_ref[0, 0, :][None, :]) * g_ref[0, 0, :][:, None]


def _combine_body(x1_ref, pr_ref, g_ref, b_ref, o_ref):
    y = x1_ref[...] + pr_ref[:, 0, :] + pr_ref[:, 1, :]
    mu = jnp.mean(y, axis=1, keepdims=True)
    yc = y - mu
    var = jnp.mean(yc * yc, axis=1, keepdims=True)
    o_ref[...] = yc * _rsqrt(var + 1e-5) * g_ref[0, :][None, :] + b_ref[0, :][None, :]


def kernel(x, Wqkv, bqkv, Wo, bo, ln1_g, ln1_b, ln2_g, ln2_b, wg, We1, be1, We2, be2):
    f32 = jnp.float32
    xf = x.reshape(T, D)

    qkv = pl.pallas_call(
        _qkv_body,
        grid=(6, 8),
        in_specs=[
            pl.BlockSpec((T // 8, D), lambda n, m: (m, 0)),
            pl.BlockSpec((3 * D // 6, D), lambda n, m: (n, 0)),
            pl.BlockSpec((1, 3 * D // 6), lambda n, m: (0, n)),
        ],
        out_specs=pl.BlockSpec((T // 8, 3 * D // 6), lambda n, m: (m, n)),
        out_shape=jax.ShapeDtypeStruct((T, 3 * D), f32),
    )(xf, Wqkv, bqkv.reshape(1, 3 * D))

    attn = pl.pallas_call(
        _attn_body,
        grid=(H // 2, 4),
        in_specs=[
            pl.BlockSpec((T // 4, 2 * DH), lambda h, m: (m, h)),
            pl.BlockSpec((T, 2 * DH), lambda h, m: (0, H // 2 + h)),
            pl.BlockSpec((T, 2 * DH), lambda h, m: (0, H + h)),
        ],
        out_specs=pl.BlockSpec((T // 4, 2 * DH), lambda h, m: (m, h)),
        out_shape=jax.ShapeDtypeStruct((T, D), f32),
    )(qkv, qkv, qkv)

    wg_pad = jnp.zeros((EPAD, D), f32).at[:E].set(wg)
    x1, ti, gt, st = pl.pallas_call(
        _ln1_router_body,
        grid=(8,),
        in_specs=[
            pl.BlockSpec((T // 8, D), lambda m: (m, 0)),
            pl.BlockSpec((D, D), lambda m: (0, 0)),
            pl.BlockSpec((1, D), lambda m: (0, 0)),
            pl.BlockSpec((T // 8, D), lambda m: (m, 0)),
            pl.BlockSpec((1, D), lambda m: (0, 0)),
            pl.BlockSpec((1, D), lambda m: (0, 0)),
            pl.BlockSpec((EPAD, D), lambda m: (0, 0)),
        ],
        out_specs=[
            pl.BlockSpec((T // 8, D), lambda m: (m, 0)),
            pl.BlockSpec((T // 8, EPAD), lambda m: (m, 0)),
            pl.BlockSpec((T // 8, EPAD), lambda m: (m, 0)),
            pl.BlockSpec((8, EPAD), lambda m: (0, 0)),
        ],
        out_shape=[
            jax.ShapeDtypeStruct((T, D), f32),
            jax.ShapeDtypeStruct((T, EPAD), jnp.int32),
            jax.ShapeDtypeStruct((T, EPAD), f32),
            jax.ShapeDtypeStruct((8, EPAD), f32),
        ],
    )(attn, Wo, bo.reshape(1, D), xf, ln1_g.reshape(1, D), ln1_b.reshape(1, D),
      wg_pad)

    topi = ti[:, :K]
    gates = gt[:, :K]
    aux = f32(E) * jnp.sum((st[1, :E] / T) * (st[0, :E] / T))

    e_flat = topi.reshape(-1)
    onehot = (e_flat[:, None] == jnp.arange(E, dtype=jnp.int32)[None, :]).astype(jnp.int32)
    rank = jnp.sum((jnp.cumsum(onehot, axis=0) - onehot) * onehot, axis=1)
    counts = jnp.sum(onehot, axis=0)
    nblk = (counts + BLK - 1) // BLK
    bstart = jnp.concatenate([jnp.zeros(1, jnp.int32), jnp.cumsum(nblk)])[:E]
    pos = (bstart * BLK)[e_flat] + rank
    tok_of_pair = jnp.arange(T * K, dtype=jnp.int32) // K
    row_token = jnp.zeros(P, jnp.int32).at[pos].set(tok_of_pair)
    row_gate = jnp.zeros(P, f32).at[pos].set(gates.reshape(-1))
    block_expert = (jnp.sum(jnp.arange(NB, dtype=jnp.int32)[:, None] >= bstart[None, :],
                            axis=1) - 1).astype(jnp.int32)

    xg = _sc_row_gather(P, 4)(x1, row_token)

    moe = pl.pallas_call(
        _ffn_body,
        grid_spec=pltpu.PrefetchScalarGridSpec(
            num_scalar_prefetch=1,
            grid=(NB, NF),
            in_specs=[
                pl.BlockSpec((BLK, D), lambda b, f, be: (b, 0)),
                pl.BlockSpec((1, FT, D), lambda b, f, be: (be[b], f, 0)),
                pl.BlockSpec((1, 1, FT), lambda b, f, be: (be[b], 0, f)),
                pl.BlockSpec((1, D, FT), lambda b, f, be: (be[b], 0, f)),
                pl.BlockSpec((1, 1, D), lambda b, f, be: (be[b], 0, 0)),
                pl.BlockSpec((1, 1, BLK), lambda b, f, be: (b, 0, 0)),
            ],
            out_specs=pl.BlockSpec((BLK, D), lambda b, f, be: (b, 0)),
        ),
        out_shape=jax.ShapeDtypeStruct((P, D), f32),
    )(block_expert, xg, We1, be1.reshape(E, 1, F), We2, be2.reshape(E, 1, D),
      row_gate.reshape(NB, 1, BLK))

    pairs = _sc_row_gather(T * K, 2)(moe, pos.astype(jnp.int32))
    out = pl.pallas_call(
        _combine_body,
        grid=(T // GC,),
        in_specs=[
            pl.BlockSpec((GC, D), lambda j: (j, 0)),
            pl.BlockSpec((GC, K, D), lambda j: (j, 0, 0)),
            pl.BlockSpec((1, D), lambda j: (0, 0)),
            pl.BlockSpec((1, D), lambda j: (0, 0)),
        ],
        out_specs=pl.BlockSpec((GC, D), lambda j: (j, 0)),
        out_shape=jax.ShapeDtypeStruct((T, D), f32),
    )(x1, pairs.reshape(T, K, D), ln2_g.reshape(1, D), ln2_b.reshape(1, D))

    return (out.reshape(1, T, D), aux)


# speedup vs baseline: 1.8174x; 1.2033x over previous
import functools

import jax
import jax.numpy as jnp
from jax import lax
from jax.experimental import pallas as pl
from jax.experimental.pallas import tpu as pltpu, tpu_sc as plsc

T, D, H, DH, F, E, K = 2048, 1024, 16, 64, 4096, 8, 2
BLK = 256
NB = (T * K) // BLK + E
P = NB * BLK
EPAD = 128
FT = 1024
NF = F // FT
GC = 256
NWORK = 32


def _sc_row_gather(B, nchunks):
    b_per_w = B // NWORK
    cb = b_per_w // nchunks
    mesh = plsc.VectorSubcoreMesh(core_axis_name="c", subcore_axis_name="s",
                                  num_cores=2, num_subcores=16)

    @functools.partial(
        pl.kernel, mesh=mesh,
        out_type=jax.ShapeDtypeStruct((B, D), jnp.float32),
        scratch_types=[
            pltpu.VMEM((cb,), jnp.int32),
            pltpu.VMEM((cb, D), jnp.float32),
            pltpu.SemaphoreType.DMA,
        ],
    )
    def k(table_hbm, idx_hbm, out_hbm, idx_v, rows_v, sem):
        wid = lax.axis_index("s") * 2 + lax.axis_index("c")
        for chunk in range(nchunks):
            base = wid * b_per_w + chunk * cb
            pltpu.sync_copy(idx_hbm.at[pl.ds(base, cb)], idx_v)
            pltpu.async_copy(table_hbm.at[idx_v], rows_v, sem).wait()
            pltpu.sync_copy(rows_v, out_hbm.at[pl.ds(base, cb)])

    return k


def _qkv_body(x_ref, w_ref, b_ref, o_ref):
    o_ref[...] = jax.lax.dot_general(
        x_ref[...], w_ref[...], (((1,), (1,)), ((), ())),
        preferred_element_type=jnp.float32) + b_ref[0, :][None, :]


def _attn_body(q_ref, k_ref, v_ref, o_ref):
    scale = DH ** -0.5
    for half in range(2):
        sl = slice(half * DH, (half + 1) * DH)
        q = q_ref[:, sl]
        k = k_ref[:, sl]
        v = v_ref[:, sl]
        s = jax.lax.dot_general(q, k, (((1,), (1,)), ((), ())),
                                preferred_element_type=jnp.float32) * scale
        m = jnp.max(s, axis=1, keepdims=True)
        e = jnp.exp(s - m)
        oe = jax.lax.dot_general(
            e, v, (((1,), (0,)), ((), ())),
            preferred_element_type=jnp.float32)
        o_ref[:, sl] = oe / jnp.sum(e, axis=1, keepdims=True)


def _ln1_router_body(a_ref, wo_ref, bo_ref, x_ref, g_ref, b_ref, wg_ref,
                     x1_ref, ti_ref, gt_ref, st_ref):
    y = jax.lax.dot_general(a_ref[...], wo_ref[...], (((1,), (1,)), ((), ())),
                            preferred_element_type=jnp.float32)
    y = y + bo_ref[0, :][None, :] + x_ref[...]
    mu = jnp.mean(y, axis=1, keepdims=True)
    yc = y - mu
    var = jnp.mean(yc * yc, axis=1, keepdims=True)
    x1 = yc / jnp.sqrt(var + 1e-5) * g_ref[0, :][None, :] + b_ref[0, :][None, :]
    x1_ref[...] = x1

    logits = jax.lax.dot_general(x1, wg_ref[...], (((1,), (1,)), ((), ())),
                                 preferred_element_type=jnp.float32)
    col = jax.lax.broadcasted_iota(jnp.int32, logits.shape, 1)
    valid = col < E
    logits = jnp.where(valid, logits, jnp.float32(-1e30))
    l1 = jnp.max(logits, axis=1, keepdims=True)
    i1 = jnp.min(jnp.where(logits == l1, col, EPAD), axis=1, keepdims=True)
    lm = jnp.where(col == i1, jnp.float32(-1e30), logits)
    l2 = jnp.max(lm, axis=1, keepdims=True)
    i2 = jnp.min(jnp.where(lm == l2, col, EPAD), axis=1, keepdims=True)
    t = jnp.exp(l2 - l1)
    r = 1.0 / (1.0 + t)
    ex = jnp.where(valid, jnp.exp(logits - l1), 0.0)
    probs = ex / jnp.sum(ex, axis=1, keepdims=True)
    ti_ref[...] = jnp.where(col == 0, i1, jnp.where(col == 1, i2, 0))
    gt_ref[...] = jnp.where(col == 0, r, jnp.where(col == 1, t * r, 0.0))

    mask = ((col == i1) | (col == i2)) & valid
    psum = jnp.sum(probs, axis=0, keepdims=True)
    fsum = jnp.sum(mask.astype(jnp.float32), axis=0, keepdims=True)

    @pl.when(pl.program_id(0) == 0)
    def _():
        st_ref[...] = jnp.zeros_like(st_ref)

    st_ref[0:1, :] += psum
    st_ref[1:2, :] += fsum


def _ffn_body(be_ref, xg_ref, w1_ref, b1_ref, w2_ref, b2_ref, g_ref, o_ref):
    f = pl.program_id(1)
    h = jax.lax.dot_general(xg_ref[...], w1_ref[0], (((1,), (1,)), ((), ())),
                            preferred_element_type=jnp.float32)
    h = jnp.maximum(h + b1_ref[0, 0, :][None, :], 0.0)
    c = jax.lax.dot_general(h, w2_ref[0], (((1,), (1,)), ((), ())),
                            preferred_element_type=jnp.float32)

    @pl.when(f == 0)
    def _():
        o_ref[...] = jnp.zeros_like(o_ref)

    o_ref[...] += c

    @pl.when(f == NF - 1)
    def _():
        o_ref[...] = (o_ref[...] + b2_ref[0, 0, :][None, :]) * g_ref[0, 0, :][:, None]


def _combine_body(x1_ref, pr_ref, g_ref, b_ref, o_ref):
    y = x1_ref[...] + pr_ref[:, 0, :] + pr_ref[:, 1, :]
    mu = jnp.mean(y, axis=1, keepdims=True)
    yc = y - mu
    var = jnp.mean(yc * yc, axis=1, keepdims=True)
    o_ref[...] = yc / jnp.sqrt(var + 1e-5) * g_ref[0, :][None, :] + b_ref[0, :][None, :]


def kernel(x, Wqkv, bqkv, Wo, bo, ln1_g, ln1_b, ln2_g, ln2_b, wg, We1, be1, We2, be2):
    f32 = jnp.float32
    xf = x.reshape(T, D)

    qkv = pl.pallas_call(
        _qkv_body,
        grid=(6, 8),
        in_specs=[
            pl.BlockSpec((T // 8, D), lambda n, m: (m, 0)),
            pl.BlockSpec((3 * D // 6, D), lambda n, m: (n, 0)),
            pl.BlockSpec((1, 3 * D // 6), lambda n, m: (0, n)),
        ],
        out_specs=pl.BlockSpec((T // 8, 3 * D // 6), lambda n, m: (m, n)),
        out_shape=jax.ShapeDtypeStruct((T, 3 * D), f32),
    )(xf, Wqkv, bqkv.reshape(1, 3 * D))

    attn = pl.pallas_call(
        _attn_body,
        grid=(H // 2, 4),
        in_specs=[
            pl.BlockSpec((T // 4, 2 * DH), lambda h, m: (m, h)),
            pl.BlockSpec((T, 2 * DH), lambda h, m: (0, H // 2 + h)),
            pl.BlockSpec((T, 2 * DH), lambda h, m: (0, H + h)),
        ],
        out_specs=pl.BlockSpec((T // 4, 2 * DH), lambda h, m: (m, h)),
        out_shape=jax.ShapeDtypeStruct((T, D), f32),
    )(qkv, qkv, qkv)

    wg_pad = jnp.zeros((EPAD, D), f32).at[:E].set(wg)
    x1, ti, gt, st = pl.pallas_call(
        _ln1_router_body,
        grid=(8,),
        in_specs=[
            pl.BlockSpec((T // 8, D), lambda m: (m, 0)),
            pl.BlockSpec((D, D), lambda m: (0, 0)),
            pl.BlockSpec((1, D), lambda m: (0, 0)),
            pl.BlockSpec((T // 8, D), lambda m: (m, 0)),
            pl.BlockSpec((1, D), lambda m: (0, 0)),
            pl.BlockSpec((1, D), lambda m: (0, 0)),
            pl.BlockSpec((EPAD, D), lambda m: (0, 0)),
        ],
        out_specs=[
            pl.BlockSpec((T // 8, D), lambda m: (m, 0)),
            pl.BlockSpec((T // 8, EPAD), lambda m: (m, 0)),
            pl.BlockSpec((T // 8, EPAD), lambda m: (m, 0)),
            pl.BlockSpec((8, EPAD), lambda m: (0, 0)),
        ],
        out_shape=[
            jax.ShapeDtypeStruct((T, D), f32),
            jax.ShapeDtypeStruct((T, EPAD), jnp.int32),
            jax.ShapeDtypeStruct((T, EPAD), f32),
            jax.ShapeDtypeStruct((8, EPAD), f32),
        ],
    )(attn, Wo, bo.reshape(1, D), xf, ln1_g.reshape(1, D), ln1_b.reshape(1, D),
      wg_pad)

    topi = ti[:, :K]
    gates = gt[:, :K]
    aux = f32(E) * jnp.sum((st[1, :E] / T) * (st[0, :E] / T))

    e_flat = topi.reshape(-1)
    onehot = (e_flat[:, None] == jnp.arange(E, dtype=jnp.int32)[None, :]).astype(jnp.int32)
    rank = jnp.sum((jnp.cumsum(onehot, axis=0) - onehot) * onehot, axis=1)
    counts = jnp.sum(onehot, axis=0)
    nblk = (counts + BLK - 1) // BLK
    bstart = jnp.concatenate([jnp.zeros(1, jnp.int32), jnp.cumsum(nblk)])[:E]
    pos = (bstart * BLK)[e_flat] + rank
    tok_of_pair = jnp.arange(T * K, dtype=jnp.int32) // K
    row_token = (jnp.arange(P, dtype=jnp.int32) % T).at[pos].set(tok_of_pair)
    row_gate = jnp.zeros(P, f32).at[pos].set(gates.reshape(-1))
    block_expert = (jnp.sum(jnp.arange(NB, dtype=jnp.int32)[:, None] >= bstart[None, :],
                            axis=1) - 1).astype(jnp.int32)

    xg = _sc_row_gather(P, 4)(x1, row_token)

    moe = pl.pallas_call(
        _ffn_body,
        grid_spec=pltpu.PrefetchScalarGridSpec(
            num_scalar_prefetch=1,
            grid=(NB, NF),
            in_specs=[
                pl.BlockSpec((BLK, D), lambda b, f, be: (b, 0)),
                pl.BlockSpec((1, FT, D), lambda b, f, be: (be[b], f, 0)),
                pl.BlockSpec((1, 1, FT), lambda b, f, be: (be[b], 0, f)),
                pl.BlockSpec((1, D, FT), lambda b, f, be: (be[b], 0, f)),
                pl.BlockSpec((1, 1, D), lambda b, f, be: (be[b], 0, 0)),
                pl.BlockSpec((1, 1, BLK), lambda b, f, be: (b, 0, 0)),
            ],
            out_specs=pl.BlockSpec((BLK, D), lambda b, f, be: (b, 0)),
        ),
        out_shape=jax.ShapeDtypeStruct((P, D), f32),
    )(block_expert, xg, We1, be1.reshape(E, 1, F), We2, be2.reshape(E, 1, D),
      row_gate.reshape(NB, 1, BLK))

    pairs = _sc_row_gather(T * K, 2)(moe, pos.astype(jnp.int32))
    out = pl.pallas_call(
        _combine_body,
        grid=(T // GC,),
        in_specs=[
            pl.BlockSpec((GC, D), lambda j: (j, 0)),
            pl.BlockSpec((GC, K, D), lambda j: (j, 0, 0)),
            pl.BlockSpec((1, D), lambda j: (0, 0)),
            pl.BlockSpec((1, D), lambda j: (0, 0)),
        ],
        out_specs=pl.BlockSpec((GC, D), lambda j: (j, 0)),
        out_shape=jax.ShapeDtypeStruct((T, D), f32),
    )(x1, pairs.reshape(T, K, D), ln2_g.reshape(1, D), ln2_b.reshape(1, D))

    return (out.reshape(1, T, D), aux)
